# baseline (speedup 1.0000x reference)
import functools

import jax
import jax.numpy as jnp
from jax import lax
from jax.experimental import pallas as pl
from jax.experimental.pallas import tpu as pltpu
from jax.experimental.pallas import tpu_sc as plsc

_LANES = 16


@functools.partial(jax.jit, static_argnums=(3, 4, 5))
def _embed_call(ids_flat, token_embedding, position_embedding, B, S, D):
    NC, NS = 2, 16
    NW = NC * NS
    R = B * S
    RPW = R // NW
    C = 32
    NCH = RPW // C
    mesh = plsc.VectorSubcoreMesh(core_axis_name="c", subcore_axis_name="s")

    @functools.partial(
        pl.kernel,
        mesh=mesh,
        out_type=jax.ShapeDtypeStruct((R, D), jnp.float32),
        scratch_types=[
            pltpu.VMEM((RPW,), jnp.int32),
            pltpu.VMEM((S, D), jnp.float32),
            pltpu.VMEM((C, D), jnp.float32),
            pltpu.SemaphoreType.DMA,
        ],
    )
    def k(ids_hbm, tok_hbm, pos_hbm, out_hbm, idx_v, pos_v, buf, sem):
        wid = lax.axis_index("s") * NC + lax.axis_index("c")
        base = wid * RPW
        pltpu.sync_copy(ids_hbm.at[pl.ds(base, RPW)], idx_v)
        pltpu.sync_copy(pos_hbm, pos_v)

        def chunk(kk, _):
            pltpu.async_copy(tok_hbm.at[idx_v.at[pl.ds(kk * C, C)]], buf, sem).wait()
            p0 = lax.rem(kk * C, S)

            def row(i, p):
                for l in range(D // _LANES):
                    sl = pl.ds(l * _LANES, _LANES)
                    buf[i, sl] = buf[i, sl] + pos_v[p, sl]
                pn = p + 1
                return jnp.where(pn == S, 0, pn)

            lax.fori_loop(0, C, row, p0)
            pltpu.sync_copy(buf, out_hbm.at[pl.ds(base + kk * C, C)])
            return 0

        lax.fori_loop(0, NCH, chunk, 0)

    return k(ids_flat, token_embedding, position_embedding)


def kernel(input_ids, token_embedding, position_embedding):
    B, S = input_ids.shape
    _, D = token_embedding.shape
    out = _embed_call(
        input_ids.reshape(-1), token_embedding, position_embedding, B, S, D
    )
    return out.reshape(B, S, D)


# speedup vs baseline: 1.2294x; 1.2294x over previous
import functools

import jax
import jax.numpy as jnp
from jax import lax
from jax.experimental import pallas as pl
from jax.experimental.pallas import tpu as pltpu
from jax.experimental.pallas import tpu_sc as plsc

_LANES = 16
_NBUF = 4


@functools.partial(jax.jit, static_argnums=(3, 4, 5))
def _embed_call(ids_grp, token_embedding, position_embedding, B, S, D):
    NC, NS = 2, 16
    NW = NC * NS
    R = B * S
    RPW = R // NW
    C = 16
    NCH = RPW // C
    NR = NCH // _NBUF
    mesh = plsc.VectorSubcoreMesh(core_axis_name="c", subcore_axis_name="s")

    @functools.partial(
        pl.kernel,
        mesh=mesh,
        out_type=jax.ShapeDtypeStruct((R, D), jnp.float32),
        scratch_types=[
            pltpu.VMEM((RPW,), jnp.int32),
            pltpu.VMEM((S, D), jnp.float32),
        ]
        + [pltpu.VMEM((C, D), jnp.float32)] * _NBUF
        + [pltpu.SemaphoreType.DMA] * (2 * _NBUF),
    )
    def k(ids_hbm, tok_hbm, pos_hbm, out_hbm, idx_v, pos_v, *bs):
        bufs = bs[:_NBUF]
        gs = bs[_NBUF:2 * _NBUF]
        os_ = bs[2 * _NBUF:]
        wid = lax.axis_index("s") * NC + lax.axis_index("c")
        base = wid * RPW
        pltpu.sync_copy(ids_hbm.at[pl.ds(base, RPW)], idx_v)
        pltpu.sync_copy(pos_hbm, pos_v)

        def gather_start(kk, b):
            pltpu.async_copy(tok_hbm.at[idx_v.at[pl.ds(kk * C, C)]], bufs[b], gs[b])

        def gather_wait(kk, b):
            pltpu.make_async_copy(
                tok_hbm.at[idx_v.at[pl.ds(kk * C, C)]], bufs[b], gs[b]
            ).wait()

        def out_start(kk, b):
            pltpu.async_copy(bufs[b], out_hbm.at[pl.ds(base + kk * C, C)], os_[b])

        def out_wait(b):
            pltpu.make_async_copy(
                bufs[b], out_hbm.at[pl.ds(base, C)], os_[b]
            ).wait()

        def add_rows(kk, b):
            buf = bufs[b]
            p0 = lax.rem(kk * C, S)

            def row(i, _):
                p = p0 + i
                p = jnp.where(p >= S, p - S, p)
                for l in range(D // _LANES):
                    sl = pl.ds(l * _LANES, _LANES)
                    buf[i, sl] = buf[i, sl] + pos_v[p, sl]
                return 0

            lax.fori_loop(0, C, row, 0)

        gather_start(0, 0)
        gather_start(1, 1)
        for b in range(_NBUF):
            kk = b
            if b >= 2:
                out_wait(b - 2)
                gather_start(kk + 2, b - 2)
            else:
                gather_start(kk + 2, b + 2)
            gather_wait(kk, b)
            add_rows(kk, b)
            out_start(kk, b)

        def round_body(t, _):
            kk0 = t * _NBUF
            for b in range(_NBUF):
                kk = kk0 + b
                b2 = (b + 2) % _NBUF
                out_wait(b2)
                gather_start(kk + 2, b2)
                gather_wait(kk, b)
                add_rows(kk, b)
                out_start(kk, b)
            return 0

        lax.fori_loop(1, NR - 1, round_body, 0)

        kk0 = (NR - 1) * _NBUF
        for b in range(_NBUF):
            kk = kk0 + b
            if b < 2:
                b2 = (b + 2) % _NBUF
                out_wait(b2)
                gather_start(kk + 2, b2)
            gather_wait(kk, b)
            add_rows(kk, b)
            out_start(kk, b)
        for b in range(_NBUF):
            out_wait(b)

    return k(ids_grp, token_embedding, position_embedding)


def kernel(input_ids, token_embedding, position_embedding):
    B, S = input_ids.shape
    _, D = token_embedding.shape
    out = _embed_call(
        input_ids.reshape(-1), token_embedding, position_embedding, B, S, D
    )
    return out.reshape(B, S, D)


# speedup vs baseline: 1.7167x; 1.3964x over previous
import functools

import jax
import jax.numpy as jnp
from jax import lax
from jax.experimental import pallas as pl
from jax.experimental.pallas import tpu as pltpu
from jax.experimental.pallas import tpu_sc as plsc

_LANES = 16
_NBUF = 4


@functools.partial(jax.jit, static_argnums=(3, 4, 5))
def _embed_call(ids_flat, token_embedding, position_embedding, B, S, D):
    NC, NS = 2, 16
    NW = NC * NS
    R = B * S
    RPW = R // NW
    C = 16
    NCH = RPW // C
    NR = NCH // _NBUF
    mesh = plsc.VectorSubcoreMesh(core_axis_name="c", subcore_axis_name="s")

    @functools.partial(
        pl.kernel,
        mesh=mesh,
        out_type=jax.ShapeDtypeStruct((R, D), jnp.float32),
        scratch_types=[
            pltpu.VMEM((RPW,), jnp.int32),
            pltpu.VMEM((S, D), jnp.float32),
        ]
        + [pltpu.VMEM((C, D), jnp.float32)] * _NBUF
        + [pltpu.SemaphoreType.DMA] * (2 * _NBUF),
    )
    def k(ids_hbm, tok_hbm, pos_hbm, out_hbm, idx_v, pos_v, *bs):
        bufs = bs[:_NBUF]
        gs = bs[_NBUF:2 * _NBUF]
        os_ = bs[2 * _NBUF:]
        wid = lax.axis_index("s") * NC + lax.axis_index("c")
        base = wid * RPW
        pltpu.sync_copy(ids_hbm.at[pl.ds(base, RPW)], idx_v)
        pltpu.sync_copy(pos_hbm, pos_v)

        def gather_start(kk, b):
            pltpu.async_copy(tok_hbm.at[idx_v.at[pl.ds(kk * C, C)]], bufs[b], gs[b])

        def gather_wait(kk, b):
            pltpu.make_async_copy(
                tok_hbm.at[idx_v.at[pl.ds(kk * C, C)]], bufs[b], gs[b]
            ).wait()

        def out_start(kk, b):
            pltpu.async_copy(bufs[b], out_hbm.at[pl.ds(base + kk * C, C)], os_[b])

        def out_wait(b):
            pltpu.make_async_copy(
                bufs[b], out_hbm.at[pl.ds(base, C)], os_[b]
            ).wait()

        def add_rows(kk, b):
            buf = bufs[b]
            p0 = lax.rem(kk * C, S)

            @plsc.parallel_loop(0, C, unroll=2)
            def _(i):
                p = p0 + i
                p = jnp.where(p >= S, p - S, p)
                for l in range(D // _LANES):
                    sl = pl.ds(l * _LANES, _LANES)
                    buf[i, sl] = buf[i, sl] + pos_v[p, sl]

        def chunk_step(kk, b):
            b2 = (b + 2) % _NBUF

            @pl.when(kk >= 2)
            def _():
                out_wait(b2)

            @pl.when(kk < NCH - 2)
            def _():
                gather_start(kk + 2, b2)

            gather_wait(kk, b)
            add_rows(kk, b)
            out_start(kk, b)

        gather_start(0, 0)
        gather_start(1, 1)

        def round_body(t, _):
            for b in range(_NBUF):
                chunk_step(t * _NBUF + b, b)
            return 0

        lax.fori_loop(0, NR, round_body, 0)

        out_wait(2)
        out_wait(3)

    return k(ids_flat, token_embedding, position_embedding)


def kernel(input_ids, token_embedding, position_embedding):
    B, S = input_ids.shape
    _, D = token_embedding.shape
    out = _embed_call(
        input_ids.reshape(-1), token_embedding, position_embedding, B, S, D
    )
    return out.reshape(B, S, D)
